# Initial kernel scaffold; baseline (speedup 1.0000x reference)
import functools

import jax
import jax.numpy as jnp
from jax import lax
from jax.experimental import pallas as pl
from jax.experimental.pallas import tpu as pltpu
from jax.experimental.pallas import tpu_sc as plsc

_G = 200
_NBINS = _G * _G
_N = 25000 * 81
_NPAD = 2 ** 21
_NC = 2
_NS = 16
_NW = _NC * _NS
_PPW = _NPAD // _NW
_CH = 8192
_KCH = _PPW // _CH
_PAD_CNT = _NPAD - _N
_CLIP_HI = _G - 1 - 1e-6


def _sc_hist_body(xs_hbm, ys_hbm, out_hbm, xbuf, ybuf, hist):
  c = lax.axis_index("c")
  s = lax.axis_index("s")
  wid = c * _NS + s
  base = wid * _PPW

  zeros16 = jnp.zeros((16,), jnp.float32)

  def zero_body(i, carry):
    hist[pl.ds(i * 16, 16)] = zeros16
    return carry

  lax.fori_loop(0, _NBINS // 16, zero_body, 0)

  ones16 = jnp.ones((16,), jnp.float32)

  def chunk_body(k, carry):
    cb = base + k * _CH
    pltpu.sync_copy(xs_hbm.at[pl.ds(cb, _CH)], xbuf)
    pltpu.sync_copy(ys_hbm.at[pl.ds(cb, _CH)], ybuf)

    def group_body(g, carry2):
      xv = xbuf[pl.ds(g * 16, 16)]
      yv = ybuf[pl.ds(g * 16, 16)]
      xc = jnp.clip(xv, 0.0, _CLIP_HI)
      yc = jnp.clip(yv, 0.0, _CLIP_HI)
      xi = (xc + 0.5).astype(jnp.int32)
      yi = (yc + 0.5).astype(jnp.int32)
      idx = xi * _G + yi
      plsc.addupdate_scatter(hist, [idx], ones16)
      return carry2

    lax.fori_loop(0, _CH // 16, group_body, 0)
    return carry

  lax.fori_loop(0, _KCH, chunk_body, 0)
  pltpu.sync_copy(hist, out_hbm.at[wid])


_sc_hist = pl.kernel(
    _sc_hist_body,
    out_type=jax.ShapeDtypeStruct((_NW, _NBINS), jnp.float32),
    mesh=plsc.VectorSubcoreMesh(core_axis_name="c", subcore_axis_name="s"),
    scratch_types=[
        pltpu.VMEM((_CH,), jnp.float32),
        pltpu.VMEM((_CH,), jnp.float32),
        pltpu.VMEM((_NBINS,), jnp.float32),
    ],
)


def _finalize_body(partials_ref, grid_ref, out_ref):
  total = jnp.sum(partials_ref[...], axis=0)
  rows = lax.broadcasted_iota(jnp.int32, (_G, _G), 0)
  cols = lax.broadcasted_iota(jnp.int32, (_G, _G), 1)
  pad_fix = jnp.where((rows == 0) & (cols == 0), float(_PAD_CNT), 0.0)
  total = total - pad_fix
  prob = total / float(25000 * 80)
  out_ref[...] = jnp.where(grid_ref[...] != 0.0, 0.0, prob)


def kernel(all_points, grid):
  pts_t = all_points.T
  padded = jnp.zeros((2, _NPAD), jnp.float32).at[:, :_N].set(pts_t)
  xs = padded[0]
  ys = padded[1]
  partials = _sc_hist(xs, ys)
  partials_3d = partials.reshape(_NW, _G, _G)
  out = pl.pallas_call(
      _finalize_body,
      out_shape=jax.ShapeDtypeStruct((_G, _G), jnp.float32),
  )(partials_3d, grid)
  return out


# speedup vs baseline: 49.6459x; 49.6459x over previous
import functools

import jax
import jax.numpy as jnp
from jax import lax
from jax.experimental import pallas as pl
from jax.experimental.pallas import tpu as pltpu
from jax.experimental.pallas import tpu_sc as plsc

_G = 200
_NBINS = _G * _G
_N = 25000 * 81
_NPAD = 2 ** 21
_NC = 2
_NS = 16
_NW = _NC * _NS
_PPW = _NPAD // _NW
_CH = 8192
_KCH = _PPW // _CH
_PAD_CNT = _NPAD - _N
_CLIP_HI = _G - 1 - 1e-6


def _sc_hist_body(xs_hbm, ys_hbm, out_hbm, xbuf, ybuf, hist):
  c = lax.axis_index("c")
  s = lax.axis_index("s")
  wid = c * _NS + s
  base = wid * _PPW

  zeros16 = jnp.zeros((16,), jnp.float32)

  def zero_body(i, carry):
    hist[pl.ds(i * 16, 16)] = zeros16
    return carry

  lax.fori_loop(0, _NBINS // 16, zero_body, 0)

  ones16 = jnp.ones((16,), jnp.float32)

  def chunk_body(k, carry):
    cb = base + k * _CH
    pltpu.sync_copy(xs_hbm.at[pl.ds(cb, _CH)], xbuf)
    pltpu.sync_copy(ys_hbm.at[pl.ds(cb, _CH)], ybuf)

    def group_body(g, carry2):
      xv = xbuf[pl.ds(g * 16, 16)]
      yv = ybuf[pl.ds(g * 16, 16)]
      xc = jnp.clip(xv, 0.0, _CLIP_HI)
      yc = jnp.clip(yv, 0.0, _CLIP_HI)
      xi = (xc + 0.5).astype(jnp.int32)
      yi = (yc + 0.5).astype(jnp.int32)
      idx = xi * _G + yi
      plsc.addupdate_scatter(hist, [idx], ones16)
      return carry2

    lax.fori_loop(0, _CH // 16, group_body, 0)
    return carry

  lax.fori_loop(0, _KCH, chunk_body, 0)
  pltpu.sync_copy(hist, out_hbm.at[wid])


_sc_hist = pl.kernel(
    _sc_hist_body,
    out_type=jax.ShapeDtypeStruct((_NW, _NBINS), jnp.float32),
    mesh=plsc.VectorSubcoreMesh(core_axis_name="c", subcore_axis_name="s"),
    scratch_types=[
        pltpu.VMEM((_CH,), jnp.float32),
        pltpu.VMEM((_CH,), jnp.float32),
        pltpu.VMEM((_NBINS,), jnp.float32),
    ],
    compiler_params=pltpu.CompilerParams(needs_layout_passes=False),
)


def _finalize_body(partials_ref, grid_ref, out_ref):
  total = jnp.sum(partials_ref[...], axis=0)
  rows = lax.broadcasted_iota(jnp.int32, (_G, _G), 0)
  cols = lax.broadcasted_iota(jnp.int32, (_G, _G), 1)
  pad_fix = jnp.where((rows == 0) & (cols == 0), float(_PAD_CNT), 0.0)
  total = total - pad_fix
  prob = total / float(25000 * 80)
  out_ref[...] = jnp.where(grid_ref[...] != 0.0, 0.0, prob)


def kernel(all_points, grid):
  pts_t = all_points.T
  padded = jnp.zeros((2, _NPAD), jnp.float32).at[:, :_N].set(pts_t)
  xs = padded[0]
  ys = padded[1]
  partials = _sc_hist(xs, ys)
  partials_3d = partials.reshape(_NW, _G, _G)
  out = pl.pallas_call(
      _finalize_body,
      out_shape=jax.ShapeDtypeStruct((_G, _G), jnp.float32),
  )(partials_3d, grid)
  return out
